# baseline (speedup 1.0000x reference)
import functools

import jax
import jax.numpy as jnp
from jax import lax
from jax.experimental import pallas as pl
from jax.experimental.pallas import tpu as pltpu
from jax.experimental.pallas import tpu_sc as plsc

VOCAB = 1000000
EMBED = 64
B = 4096

_info = plsc.get_sparse_core_info()
_NC, _NS = _info.num_cores, _info.num_subcores
_NW = _NC * _NS
_BPW = B // _NW


def _make_gather():
    mesh = plsc.VectorSubcoreMesh(core_axis_name="c", subcore_axis_name="s")

    @functools.partial(
        pl.kernel,
        mesh=mesh,
        out_type=[
            jax.ShapeDtypeStruct((B, EMBED), jnp.float32),
            jax.ShapeDtypeStruct((B, EMBED), jnp.float32),
        ],
        scratch_types=[
            pltpu.VMEM((_BPW,), jnp.int32),
            pltpu.VMEM((_BPW,), jnp.int32),
            pltpu.VMEM((_BPW, EMBED), jnp.float32),
            pltpu.VMEM((_BPW, EMBED), jnp.float32),
            pltpu.SemaphoreType.DMA,
            pltpu.SemaphoreType.DMA,
        ],
        compiler_params=pltpu.CompilerParams(use_tc_tiling_on_sc=False),
    )
    def gather_k(tgt_hbm, ctx_hbm, in_tab, out_tab, ine_hbm, oute_hbm,
                 idx_a, idx_b, rows_a, rows_b, sem_a, sem_b):
        wid = lax.axis_index("s") * _NC + lax.axis_index("c")
        base = wid * _BPW
        pltpu.sync_copy(tgt_hbm.at[pl.ds(base, _BPW)], idx_a)
        pltpu.sync_copy(ctx_hbm.at[pl.ds(base, _BPW)], idx_b)
        cp_a = pltpu.async_copy(in_tab.at[idx_a], rows_a, sem_a)
        cp_b = pltpu.async_copy(out_tab.at[idx_b], rows_b, sem_b)
        cp_a.wait()
        pltpu.sync_copy(rows_a, ine_hbm.at[pl.ds(base, _BPW)])
        cp_b.wait()
        pltpu.sync_copy(rows_b, oute_hbm.at[pl.ds(base, _BPW)])

    return gather_k


_gather = _make_gather()

_BM = 256


def _mm_body(a_ref, b_ref, o_ref):
    o_ref[...] = lax.dot_general(
        a_ref[...], b_ref[...], (((1,), (1,)), ((), ())),
        preferred_element_type=jnp.float32)


def kernel(target, context, in_embed, out_embed):
    tgt = target.astype(jnp.int32)
    ctx = context.astype(jnp.int32)
    in_embeds, out_embeds = _gather(tgt, ctx, in_embed, out_embed)
    scores = pl.pallas_call(
        _mm_body,
        grid=(B // _BM,),
        in_specs=[
            pl.BlockSpec((_BM, EMBED), lambda i: (i, 0)),
            pl.BlockSpec((B, EMBED), lambda i: (0, 0)),
        ],
        out_specs=pl.BlockSpec((_BM, B), lambda i: (i, 0)),
        out_shape=jax.ShapeDtypeStruct((B, B), jnp.float32),
    )(in_embeds, out_embeds)
    return scores


# speedup vs baseline: 1.5661x; 1.5661x over previous
import functools

import jax
import jax.numpy as jnp
from jax import lax
from jax.experimental import pallas as pl
from jax.experimental.pallas import tpu as pltpu
from jax.experimental.pallas import tpu_sc as plsc

VOCAB = 1000000
EMBED = 64
B = 4096

_info = plsc.get_sparse_core_info()
_NC, _NS = _info.num_cores, _info.num_subcores
_NW = _NC * _NS
_BPW = B // _NW
_CHUNK = 16


def _make_gather():
    mesh = plsc.VectorSubcoreMesh(core_axis_name="c", subcore_axis_name="s")

    @functools.partial(
        pl.kernel,
        mesh=mesh,
        out_type=[
            jax.ShapeDtypeStruct((B, EMBED), jnp.float32),
            jax.ShapeDtypeStruct((B, EMBED), jnp.float32),
        ],
        scratch_types=[
            pltpu.VMEM((_BPW,), jnp.int32),
            pltpu.VMEM((_BPW,), jnp.int32),
            pltpu.VMEM((_BPW, EMBED), jnp.float32),
            pltpu.VMEM((_BPW, EMBED), jnp.float32),
            pltpu.SemaphoreType.DMA,
            pltpu.SemaphoreType.DMA,
        ],
    )
    def gather_k(tgt_hbm, ctx_hbm, in_tab, out_tab, ine_hbm, oute_hbm,
                 idx_va, idx_vb, rows_a, rows_b, sem_a, sem_b):
        wid = lax.axis_index("s") * _NC + lax.axis_index("c")
        base = wid * _BPW
        pltpu.sync_copy(tgt_hbm.at[pl.ds(base, _BPW)], idx_va)
        pltpu.sync_copy(ctx_hbm.at[pl.ds(base, _BPW)], idx_vb)

        def chunk_body(c, carry):
            j0 = c * _CHUNK
            veca = idx_va[pl.ds(j0, _CHUNK)]
            vecb = idx_vb[pl.ds(j0, _CHUNK)]
            for i in range(_CHUNK):
                j = j0 + i
                ra = veca[i]
                pltpu.async_copy(in_tab.at[pl.ds(ra, 1)],
                                 rows_a.at[pl.ds(j, 1)], sem_a)
                rb = vecb[i]
                pltpu.async_copy(out_tab.at[pl.ds(rb, 1)],
                                 rows_b.at[pl.ds(j, 1)], sem_b)
            for i in range(_CHUNK):
                j = j0 + i
                pltpu.make_async_copy(in_tab.at[pl.ds(0, 1)],
                                      rows_a.at[pl.ds(j, 1)], sem_a).wait()
                pltpu.make_async_copy(out_tab.at[pl.ds(0, 1)],
                                      rows_b.at[pl.ds(j, 1)], sem_b).wait()
            return carry

        lax.fori_loop(0, _BPW // _CHUNK, chunk_body, 0)
        pltpu.sync_copy(rows_a, ine_hbm.at[pl.ds(base, _BPW)])
        pltpu.sync_copy(rows_b, oute_hbm.at[pl.ds(base, _BPW)])

    return gather_k


_gather = _make_gather()

_BM = 512


def _mm_body(a_ref, b_ref, o_ref):
    a = a_ref[...].astype(jnp.bfloat16)
    b = b_ref[...].astype(jnp.bfloat16)
    o_ref[...] = lax.dot_general(
        a, b, (((1,), (1,)), ((), ())),
        preferred_element_type=jnp.float32)


def kernel(target, context, in_embed, out_embed):
    tgt = target.astype(jnp.int32)
    ctx = context.astype(jnp.int32)
    in_embeds, out_embeds = _gather(tgt, ctx, in_embed, out_embed)
    scores = pl.pallas_call(
        _mm_body,
        grid=(B // _BM,),
        in_specs=[
            pl.BlockSpec((_BM, EMBED), lambda i: (i, 0)),
            pl.BlockSpec((B, EMBED), lambda i: (0, 0)),
        ],
        out_specs=pl.BlockSpec((_BM, B), lambda i: (i, 0)),
        out_shape=jax.ShapeDtypeStruct((B, B), jnp.float32),
    )(in_embeds, out_embeds)
    return scores
